# baseline (device time: 102894 ns/iter reference)
import os

import jax
import jax.numpy as jnp
from jax import lax
from jax.experimental import pallas as pl
from jax.experimental.pallas import tpu as pltpu

N_DEV = 16
N_GROUPS = 4
K_PIECES = 4
_GELU_C = 0.7978845608028654
_NOSEND = os.environ.get("A2A_NOSEND", "0") == "1"


def _gelu(y):
    return 0.5 * y * (1.0 + jnp.tanh(_GELU_C * (y + 0.044715 * y * y * y)))


def kernel(x, w_mat):
    m_per, k = x.shape
    _, n = w_mat.shape
    n_per = n // N_DEV
    m_out = N_DEV * m_per
    gcols = n // N_GROUPS
    krows = k // K_PIECES
    n_idx = N_GROUPS * K_PIECES

    def body(x_ref, w_hbm, out_ref, w_buf, y_buf, r_buf,
             load_sems, send_sems, recv_sems):
        me = lax.axis_index("i")
        my_block = lax.div(me, 4)

        barrier = pltpu.get_barrier_semaphore()
        for d in range(N_DEV):
            @pl.when(me != d)
            def _():
                pl.semaphore_signal(
                    barrier, inc=1,
                    device_id=(d,), device_id_type=pl.DeviceIdType.MESH,
                )
        pl.semaphore_wait(barrier, N_DEV - 1)

        def gidx_of(c):
            return lax.rem(my_block + c, N_GROUPS)

        def w_piece_copy(idx, slot):
            c, kr = idx // K_PIECES, idx % K_PIECES
            return pltpu.make_async_copy(
                w_hbm.at[pl.ds(kr * krows, krows),
                         pl.ds(gidx_of(c) * gcols, gcols)],
                w_buf.at[slot],
                load_sems.at[slot],
            )

        w_piece_copy(0, 0).start()
        w_piece_copy(1, 1).start()

        for c in range(N_GROUPS):
            gidx = gidx_of(c)
            y_acc = None
            for kr in range(K_PIECES):
                idx = c * K_PIECES + kr
                slot = idx % 2
                w_piece_copy(idx, slot).wait()
                part = jnp.dot(x_ref[:, kr * krows:(kr + 1) * krows],
                               w_buf[slot],
                               preferred_element_type=jnp.float32)
                y_acc = part if y_acc is None else y_acc + part
                if idx + 2 < n_idx:
                    w_piece_copy(idx + 2, slot).start()
            y = _gelu(y_acc)

            for t in range(N_GROUPS):
                j = gidx * 4 + t
                sidx = c * 4 + t
                tile = y[:, t * n_per:(t + 1) * n_per]

                @pl.when(j == me)
                def _():
                    out_ref[pl.ds(me * m_per, m_per), :] = tile

                if not _NOSEND:
                    @pl.when(j != me)
                    def _():
                        y_buf[sidx] = tile.astype(jnp.bfloat16)
                        pltpu.make_async_remote_copy(
                            src_ref=y_buf.at[sidx],
                            dst_ref=r_buf.at[me],
                            send_sem=send_sems.at[sidx],
                            recv_sem=recv_sems.at[me],
                            device_id=(j,),
                            device_id_type=pl.DeviceIdType.MESH,
                        ).start()

        if not _NOSEND:
            for c in range(N_GROUPS):
                src_block = lax.rem(my_block - c + N_GROUPS, N_GROUPS)
                for t in range(N_GROUPS):
                    s = src_block * 4 + t

                    @pl.when(s != me)
                    def _():
                        recv = pltpu.make_async_remote_copy(
                            src_ref=y_buf.at[0],
                            dst_ref=r_buf.at[s],
                            send_sem=send_sems.at[0],
                            recv_sem=recv_sems.at[s],
                            device_id=(s,),
                            device_id_type=pl.DeviceIdType.MESH,
                        )
                        recv.wait_recv()
                        out_ref[pl.ds(s * m_per, m_per), :] = (
                            r_buf[s].astype(jnp.float32))

            for c in range(N_GROUPS):
                gidx = gidx_of(c)
                for t in range(N_GROUPS):
                    j = gidx * 4 + t
                    sidx = c * 4 + t

                    @pl.when(j != me)
                    def _():
                        pltpu.make_async_remote_copy(
                            src_ref=y_buf.at[sidx],
                            dst_ref=r_buf.at[me],
                            send_sem=send_sems.at[sidx],
                            recv_sem=recv_sems.at[me],
                            device_id=(j,),
                            device_id_type=pl.DeviceIdType.MESH,
                        ).wait_send()

    return pl.pallas_call(
        body,
        out_shape=jax.ShapeDtypeStruct((m_out, n_per), jnp.float32),
        in_specs=[
            pl.BlockSpec(memory_space=pltpu.VMEM),
            pl.BlockSpec(memory_space=pl.ANY),
        ],
        out_specs=pl.BlockSpec(memory_space=pltpu.VMEM),
        scratch_shapes=[
            pltpu.VMEM((2, krows, gcols), w_mat.dtype),
            pltpu.VMEM((N_DEV, m_per, n_per), jnp.bfloat16),
            pltpu.VMEM((N_DEV, m_per, n_per), jnp.bfloat16),
            pltpu.SemaphoreType.DMA((2,)),
            pltpu.SemaphoreType.DMA((N_DEV,)),
            pltpu.SemaphoreType.DMA((N_DEV,)),
        ],
        compiler_params=pltpu.CompilerParams(
            collective_id=0,
            vmem_limit_bytes=60 * 1024 * 1024,
        ),
    )(x, w_mat)
